# baseline (speedup 1.0000x reference)
import functools

import jax
import jax.numpy as jnp
from jax import lax
from jax.experimental import pallas as pl
from jax.experimental.pallas import tpu as pltpu
from jax.experimental.pallas import tpu_sc as plsc

B = 16384
H = 64
NT = 4
CH = 128
BM = 512

_HI = jax.lax.Precision.HIGHEST


def _sc_gather_body(nw, bpw, nchunk,
                    idx_hbm, t0, t1, t2, t3, o0, o1, o2, o3,
                    idx_v, rows_v, sem):
    c = lax.axis_index("c")
    s = lax.axis_index("s")
    wid = s * 2 + c
    base = wid * bpw
    pltpu.sync_copy(idx_hbm.at[wid], idx_v)
    tabs = (t0, t1, t2, t3)
    outs = (o0, o1, o2, o3)
    for t in range(NT):
        descs = [
            pltpu.async_copy(
                tabs[t].at[idx_v.at[t, j]],
                rows_v.at[pl.ds(j * CH, CH)],
                sem,
            )
            for j in range(nchunk)
        ]
        for d in descs:
            d.wait()
        pltpu.sync_copy(rows_v, outs[t].at[pl.ds(base, bpw)])


def _gather_embeddings(idx, emb_p, emb_l1, emb_l2, emb_l3):
    info = plsc.get_sparse_core_info()
    nc, ns = info.num_cores, info.num_subcores
    nw = nc * ns
    bpw = B // nw
    nchunk = bpw // CH
    idx_r = jnp.transpose(idx.reshape(NT, nw, nchunk, CH), (1, 0, 2, 3))
    mesh = plsc.VectorSubcoreMesh(core_axis_name="c", subcore_axis_name="s")
    out_t = [jax.ShapeDtypeStruct((B, H), jnp.float32)] * NT
    body = functools.partial(_sc_gather_body, nw, bpw, nchunk)
    k = pl.kernel(
        body,
        out_type=out_t,
        mesh=mesh,
        scratch_types=[
            pltpu.VMEM((NT, nchunk, CH), jnp.int32),
            pltpu.VMEM((bpw, H), jnp.float32),
            pltpu.SemaphoreType.DMA,
        ],
        compiler_params=pltpu.CompilerParams(use_tc_tiling_on_sc=False),
    )
    return k(idx_r, emb_p, emb_l1, emb_l2, emb_l3)


def _tc_body(ep, e1, e2, e3, w1t, b1, w2t, b2, wpot, bpo, wppt, bpp,
             wvt, bv, wvp, bvp, po_ref, v_ref):
    x = (jnp.dot(ep[:], w1t[0:H, :], precision=_HI)
         + jnp.dot(e1[:], w1t[H:2 * H, :], precision=_HI)
         + jnp.dot(e2[:], w1t[2 * H:3 * H, :], precision=_HI)
         + jnp.dot(e3[:], w1t[3 * H:4 * H, :], precision=_HI))
    r1 = jnp.maximum(x + b1[:], 0.0)
    r2 = jnp.maximum(jnp.dot(r1, w2t[:], precision=_HI) + b2[:], 0.0)
    rpo = jnp.maximum(jnp.dot(r2, wpot[:], precision=_HI) + bpo[:], 0.0)
    logits = jnp.dot(rpo, wppt[:], precision=_HI) + bpp[:]
    m = jnp.max(logits, axis=1, keepdims=True)
    lse = jnp.log(jnp.sum(jnp.exp(logits - m), axis=1, keepdims=True)) + m
    po_ref[:] = logits - lse
    rv = jnp.maximum(jnp.dot(r2, wvt[:], precision=_HI) + bv[:], 0.0)
    v_ref[:] = jnp.tanh(jnp.sum(rv * wvp[:], axis=1, keepdims=True) + bvp[:])


def kernel(x_p, x_l, emb_p, emb_l1, emb_l2, emb_l3, W1, b1, W2, b2,
           Wpo, bpo, Wpp, bpp, Wv, bv, Wvp, bvp):
    P = Wpo.shape[0]
    V = Wpp.shape[0]
    idx = jnp.concatenate(
        [x_p.astype(jnp.int32), x_l.astype(jnp.int32)], axis=1).T
    ep, e1, e2, e3 = _gather_embeddings(idx, emb_p, emb_l1, emb_l2, emb_l3)

    w1t = W1.T
    w2t = W2.T
    wpot = Wpo.T
    wppt = Wpp.T
    wvt = Wv.T
    grid = (B // BM,)
    row_spec = pl.BlockSpec((BM, H), lambda i: (i, 0))
    full = lambda a: pl.BlockSpec(a.shape, lambda i: (0,) * a.ndim)
    b1r = b1.reshape(1, H)
    b2r = b2.reshape(1, H)
    bpor = bpo.reshape(1, P)
    bppr = bpp.reshape(1, V)
    bvr = bv.reshape(1, P)
    wvpr = Wvp.reshape(1, P)
    bvpr = bvp.reshape(1, 1)
    po, v = pl.pallas_call(
        _tc_body,
        grid=grid,
        in_specs=[row_spec, row_spec, row_spec, row_spec,
                  full(w1t), full(b1r), full(w2t), full(b2r),
                  full(wpot), full(bpor), full(wppt), full(bppr),
                  full(wvt), full(bvr), full(wvpr), full(bvpr)],
        out_specs=[pl.BlockSpec((BM, V), lambda i: (i, 0)),
                   pl.BlockSpec((BM, 1), lambda i: (i, 0))],
        out_shape=[jax.ShapeDtypeStruct((B, V), jnp.float32),
                   jax.ShapeDtypeStruct((B, 1), jnp.float32)],
        compiler_params=pltpu.CompilerParams(
            dimension_semantics=("arbitrary",)),
    )(ep, e1, e2, e3, w1t, b1r, w2t, b2r, wpot, bpor, wppt, bppr,
      wvt, bvr, wvpr, bvpr)
    return (po, v)
